# baseline (speedup 1.0000x reference)
import jax
import jax.numpy as jnp
from jax.experimental import pallas as pl
from jax.experimental.pallas import tpu as pltpu

_BLK = 400


def _body(x1_ref, x2_ref, w_ref, b_ref, adja_ref, adjb_ref, msk_ref,
          mskrow_ref, a_ref, dw_ref, db_ref, sb1_ref, sb2_ref, out_ref,
          fts_ref, h12_ref, colsum_ref):
    i = pl.program_id(0)
    nb = pl.num_programs(0)
    nh = w_ref.shape[1]

    @pl.when(i == 0)
    def _prologue():
        w = w_ref[...]
        b = b_ref[...]
        f1 = jnp.dot(x1_ref[...], w, preferred_element_type=jnp.float32) + b
        f2 = jnp.dot(x2_ref[...], w, preferred_element_type=jnp.float32) + b
        fts_ref[...] = jnp.concatenate([f1, f2], axis=1).astype(jnp.bfloat16)
        colsum_ref[...] = jnp.zeros_like(colsum_ref)

    fts = fts_ref[...]
    pre_a = jnp.dot(adja_ref[...], fts, preferred_element_type=jnp.float32)
    pre_b = jnp.dot(adjb_ref[...], fts, preferred_element_type=jnp.float32)
    pre = jnp.concatenate([pre_a, pre_b], axis=0)
    a = a_ref[0, 0]
    h = jnp.where(pre > 0, pre, a * pre)
    blk = 2 * adja_ref.shape[0]
    h12_ref[pl.ds(i * blk, blk), :] = h.astype(jnp.bfloat16)
    h1m = h[:, :nh] * msk_ref[...]
    colsum_ref[...] += jnp.sum(h1m, axis=0, keepdims=True)

    @pl.when(i == nb - 1)
    def _epilogue():
        msksum = jnp.sum(mskrow_ref[...])
        c = jax.nn.sigmoid(colsum_ref[...] / msksum)
        v = jax.lax.dot_general(c, dw_ref[...], (((1,), (1,)), ((), ())),
                                preferred_element_type=jnp.float32)
        db = db_ref[0, 0]
        vb = v.astype(jnp.bfloat16)
        s1 = jax.lax.dot_general(vb, h12_ref[:, :nh],
                                 (((1,), (1,)), ((), ())),
                                 preferred_element_type=jnp.float32)
        s2 = jax.lax.dot_general(vb, h12_ref[:, nh:],
                                 (((1,), (1,)), ((), ())),
                                 preferred_element_type=jnp.float32)
        sb = jnp.concatenate([sb1_ref[...], sb2_ref[...]], axis=0)
        out_ref[...] = jnp.concatenate([s1, s2], axis=0) + sb + db


def kernel(x_1, x_2, adj, sparse, msk, samp_bias1, samp_bias2, W, b_gcn,
           prelu_a, disc_W, disc_b):
    n = x_1.shape[1]
    n_in = x_1.shape[2]
    n_h = W.shape[1]
    x1 = x_1.reshape(n, n_in)
    x2 = x_2.reshape(n, n_in)
    adj2 = adj.reshape(n, n)
    b2 = jnp.asarray(b_gcn, jnp.float32).reshape(1, n_h)
    a2 = jnp.asarray(prelu_a, jnp.float32).reshape(1, 1)
    db2 = jnp.asarray(disc_b, jnp.float32).reshape(1, 1)
    msk_col = msk.reshape(n, 1)
    msk_row = msk.reshape(1, n)

    nb = n // _BLK
    out = pl.pallas_call(
        _body,
        grid=(nb,),
        in_specs=[
            pl.BlockSpec((n, n_in), lambda i: (0, 0)),
            pl.BlockSpec((n, n_in), lambda i: (0, 0)),
            pl.BlockSpec((n_in, n_h), lambda i: (0, 0)),
            pl.BlockSpec((1, n_h), lambda i: (0, 0)),
            pl.BlockSpec((_BLK // 2, n), lambda i: (2 * i, 0)),
            pl.BlockSpec((_BLK // 2, n), lambda i: (2 * i + 1, 0)),
            pl.BlockSpec((_BLK, 1), lambda i: (i, 0)),
            pl.BlockSpec((1, n), lambda i: (0, 0)),
            pl.BlockSpec((1, 1), lambda i: (0, 0)),
            pl.BlockSpec((n_h, n_h), lambda i: (0, 0)),
            pl.BlockSpec((1, 1), lambda i: (0, 0)),
            pl.BlockSpec((1, n), lambda i: (0, 0)),
            pl.BlockSpec((1, n), lambda i: (0, 0)),
        ],
        out_specs=pl.BlockSpec((2, n), lambda i: (0, 0)),
        out_shape=jax.ShapeDtypeStruct((2, n), jnp.float32),
        scratch_shapes=[
            pltpu.VMEM((n, 2 * n_h), jnp.bfloat16),
            pltpu.VMEM((n, 2 * n_h), jnp.bfloat16),
            pltpu.VMEM((1, n_h), jnp.float32),
        ],
        compiler_params=pltpu.CompilerParams(
            dimension_semantics=("arbitrary",)),
    )(x1, x2, W, b2, adj2, adj2, msk_col, msk_row, a2, disc_W, db2,
      samp_bias1, samp_bias2)

    return out.reshape(1, 2 * n)


# speedup vs baseline: 1.0102x; 1.0102x over previous
import jax
import jax.numpy as jnp
from jax.experimental import pallas as pl
from jax.experimental.pallas import tpu as pltpu

_BLK = 400


def _body(x1_ref, x2_ref, w_ref, b_ref, adja_ref, adjb_ref, msk_ref,
          mskrow_ref, a_ref, dw_ref, db_ref, sb1_ref, sb2_ref, out_ref,
          fts_ref, h12_ref, colsum_ref):
    i = pl.program_id(0)
    nb = pl.num_programs(0)
    nh = w_ref.shape[1]

    @pl.when(i == 0)
    def _prologue():
        w = w_ref[...]
        b = b_ref[...]
        f1 = jnp.dot(x1_ref[...], w, preferred_element_type=jnp.float32) + b
        f2 = jnp.dot(x2_ref[...], w, preferred_element_type=jnp.float32) + b
        fts_ref[...] = jnp.concatenate([f1, f2], axis=1).astype(jnp.bfloat16)
        colsum_ref[...] = jnp.zeros_like(colsum_ref)

    fts = fts_ref[...]
    pre_a = jnp.dot(adja_ref[...], fts, preferred_element_type=jnp.float32)
    pre_b = jnp.dot(adjb_ref[...], fts, preferred_element_type=jnp.float32)
    pre = jnp.concatenate([pre_a, pre_b], axis=0)
    a = a_ref[0, 0]
    h = jnp.where(pre > 0, pre, a * pre)
    blk = 2 * adja_ref.shape[0]
    h12_ref[pl.ds(i * blk, blk), :] = h.astype(jnp.bfloat16)
    h1m = h[:, :nh] * msk_ref[...]
    colsum_ref[...] += jnp.sum(h1m, axis=0, keepdims=True)

    @pl.when(i == nb - 1)
    def _epilogue():
        msksum = jnp.sum(mskrow_ref[...])
        c = jax.nn.sigmoid(colsum_ref[...] / msksum)
        v = jax.lax.dot_general(c, dw_ref[...], (((1,), (1,)), ((), ())),
                                preferred_element_type=jnp.float32)
        db = db_ref[0, 0]
        vb = v.astype(jnp.bfloat16)
        s1 = jax.lax.dot_general(vb, h12_ref[:, :nh],
                                 (((1,), (1,)), ((), ())),
                                 preferred_element_type=jnp.float32)
        s2 = jax.lax.dot_general(vb, h12_ref[:, nh:],
                                 (((1,), (1,)), ((), ())),
                                 preferred_element_type=jnp.float32)
        out_ref[...] = jnp.concatenate(
            [s1 + sb1_ref[...], s2 + sb2_ref[...]], axis=1) + db


def kernel(x_1, x_2, adj, sparse, msk, samp_bias1, samp_bias2, W, b_gcn,
           prelu_a, disc_W, disc_b):
    n = x_1.shape[1]
    n_in = x_1.shape[2]
    n_h = W.shape[1]
    x1 = x_1.reshape(n, n_in)
    x2 = x_2.reshape(n, n_in)
    adj2 = adj.reshape(n, n)
    b2 = jnp.asarray(b_gcn, jnp.float32).reshape(1, n_h)
    a2 = jnp.asarray(prelu_a, jnp.float32).reshape(1, 1)
    db2 = jnp.asarray(disc_b, jnp.float32).reshape(1, 1)
    msk_col = msk.reshape(n, 1)
    msk_row = msk.reshape(1, n)

    nb = n // _BLK
    out = pl.pallas_call(
        _body,
        grid=(nb,),
        in_specs=[
            pl.BlockSpec((n, n_in), lambda i: (0, 0)),
            pl.BlockSpec((n, n_in), lambda i: (0, 0)),
            pl.BlockSpec((n_in, n_h), lambda i: (0, 0)),
            pl.BlockSpec((1, n_h), lambda i: (0, 0)),
            pl.BlockSpec((_BLK // 2, n), lambda i: (2 * i, 0)),
            pl.BlockSpec((_BLK // 2, n), lambda i: (2 * i + 1, 0)),
            pl.BlockSpec((_BLK, 1), lambda i: (i, 0)),
            pl.BlockSpec((1, n), lambda i: (0, 0)),
            pl.BlockSpec((1, 1), lambda i: (0, 0)),
            pl.BlockSpec((n_h, n_h), lambda i: (0, 0)),
            pl.BlockSpec((1, 1), lambda i: (0, 0)),
            pl.BlockSpec((1, n), lambda i: (0, 0)),
            pl.BlockSpec((1, n), lambda i: (0, 0)),
        ],
        out_specs=pl.BlockSpec((1, 2 * n), lambda i: (0, 0)),
        out_shape=jax.ShapeDtypeStruct((1, 2 * n), jnp.float32),
        scratch_shapes=[
            pltpu.VMEM((n, 2 * n_h), jnp.bfloat16),
            pltpu.VMEM((n, 2 * n_h), jnp.bfloat16),
            pltpu.VMEM((1, n_h), jnp.float32),
        ],
        compiler_params=pltpu.CompilerParams(
            dimension_semantics=("arbitrary",)),
    )(x1, x2, W, b2, adj2, adj2, msk_col, msk_row, a2, disc_W, db2,
      samp_bias1, samp_bias2)

    return out
